# baseline (speedup 1.0000x reference)
import jax
import jax.numpy as jnp
from jax import lax
from jax.experimental import pallas as pl
from jax.experimental.pallas import tpu as pltpu
from jax.experimental.pallas import tpu_sc as plsc

B = 4
S = 2048
D = 1024
NC = 2
NS = 16
NW = NC * NS
SPW = S // NW
CH = 16
NCHUNK = SPW // CH
NSTAGE = NCHUNK * B
NVEC = D // 16
NXB = 5


def _sc_body(x_hbm, emb_hbm, out_hbm,
             eb0, eb1, xb0, xb1, xb2, xb3, xb4,
             esem0, esem1, ls0, ls1, ls2, ls3, ls4, ss0, ss1, ss2, ss3, ss4):
    wid = lax.axis_index("s") * NC + lax.axis_index("c")
    s0 = wid * SPW
    ebufs, esems = (eb0, eb1), (esem0, esem1)
    xbufs = (xb0, xb1, xb2, xb3, xb4)
    lsems = (ls0, ls1, ls2, ls3, ls4)
    ssems = (ss0, ss1, ss2, ss3, ss4)

    def soff(c):
        return s0 + c * CH

    def start_load(t):
        c, b = divmod(t, B)
        return pltpu.async_copy(
            x_hbm.at[b, pl.ds(soff(c), CH)], xbufs[t % NXB], lsems[t % NXB])

    eload = {0: pltpu.async_copy(emb_hbm.at[pl.ds(soff(0), CH)], eb0, esem0)}
    xload = {t: start_load(t) for t in range(3)}
    store = {}

    for t in range(NSTAGE):
        c, b = divmod(t, B)
        if b == 0 and c + 1 < NCHUNK:
            ne = (c + 1) % 2
            eload[c + 1] = pltpu.async_copy(
                emb_hbm.at[pl.ds(soff(c + 1), CH)], ebufs[ne], esems[ne])
        if t + 3 < NSTAGE:
            if t - 2 in store:
                store.pop(t - 2).wait()
            xload[t + 3] = start_load(t + 3)
        xload.pop(t).wait()
        if b == 0:
            eload.pop(c).wait()

        xbuf, ebuf = xbufs[t % NXB], ebufs[c % 2]

        def row_body(r, rc, xbuf=xbuf, ebuf=ebuf):
            for j in range(NVEC):
                sl = pl.ds(j * 16, 16)
                xbuf[r, sl] = xbuf[r, sl] + ebuf[r, sl]
            return rc

        lax.fori_loop(0, CH, row_body, 0)
        store[t] = pltpu.async_copy(
            xbuf, out_hbm.at[b, pl.ds(soff(c), CH)], ssems[t % NXB])

    for h in store.values():
        h.wait()


def kernel(x, emb_weight):
    mesh = plsc.VectorSubcoreMesh(core_axis_name="c", subcore_axis_name="s")
    f = pl.kernel(
        _sc_body,
        out_type=jax.ShapeDtypeStruct((B, S, D), jnp.float32),
        mesh=mesh,
        scratch_types=[
            pltpu.VMEM((CH, D), jnp.float32),
            pltpu.VMEM((CH, D), jnp.float32),
            pltpu.VMEM((CH, D), jnp.float32),
            pltpu.VMEM((CH, D), jnp.float32),
            pltpu.VMEM((CH, D), jnp.float32),
            pltpu.VMEM((CH, D), jnp.float32),
            pltpu.VMEM((CH, D), jnp.float32),
            pltpu.SemaphoreType.DMA,
            pltpu.SemaphoreType.DMA,
            pltpu.SemaphoreType.DMA,
            pltpu.SemaphoreType.DMA,
            pltpu.SemaphoreType.DMA,
            pltpu.SemaphoreType.DMA,
            pltpu.SemaphoreType.DMA,
            pltpu.SemaphoreType.DMA,
            pltpu.SemaphoreType.DMA,
            pltpu.SemaphoreType.DMA,
            pltpu.SemaphoreType.DMA,
            pltpu.SemaphoreType.DMA,
        ],
    )
    return f(x, emb_weight)


# speedup vs baseline: 1.1170x; 1.1170x over previous
import jax
import jax.numpy as jnp
from jax import lax
from jax.experimental import pallas as pl
from jax.experimental.pallas import tpu as pltpu
from jax.experimental.pallas import tpu_sc as plsc

B = 4
B_SC = 1
S = 2048
D = 1024
NC = 2
NS = 16
NW = NC * NS
SPW = S // NW
CH = 16
NCHUNK = SPW // CH
NSTAGE = NCHUNK * B_SC
NVEC = D // 16
NXB = 4
BS_TC = 512


def _sc_body(x_hbm, emb_hbm, out_hbm,
             eb0, eb1, xb0, xb1, xb2, xb3,
             esem0, esem1, ls0, ls1, ls2, ls3, ss0, ss1, ss2, ss3):
    wid = lax.axis_index("s") * NC + lax.axis_index("c")
    s0 = wid * SPW
    ebufs, esems = (eb0, eb1), (esem0, esem1)
    xbufs = (xb0, xb1, xb2, xb3)
    lsems = (ls0, ls1, ls2, ls3)
    ssems = (ss0, ss1, ss2, ss3)

    def soff(c):
        return s0 + c * CH

    def start_load(t):
        c, b = divmod(t, B_SC)
        return pltpu.async_copy(
            x_hbm.at[b, pl.ds(soff(c), CH)], xbufs[t % NXB], lsems[t % NXB])

    eload = {0: pltpu.async_copy(emb_hbm.at[pl.ds(soff(0), CH)], eb0, esem0)}
    xload = {t: start_load(t) for t in range(min(2, NSTAGE))}
    store = {}

    for t in range(NSTAGE):
        c, b = divmod(t, B_SC)
        if b == 0 and c + 1 < NCHUNK:
            ne = (c + 1) % 2
            eload[c + 1] = pltpu.async_copy(
                emb_hbm.at[pl.ds(soff(c + 1), CH)], ebufs[ne], esems[ne])
        if t + 2 < NSTAGE:
            if t - 2 in store:
                store.pop(t - 2).wait()
            xload[t + 2] = start_load(t + 2)
        xload.pop(t).wait()
        if b == 0:
            eload.pop(c).wait()

        xbuf, ebuf = xbufs[t % NXB], ebufs[c % 2]

        def row_body(r, rc, xbuf=xbuf, ebuf=ebuf):
            for j in range(NVEC):
                sl = pl.ds(j * 16, 16)
                xbuf[r, sl] = xbuf[r, sl] + ebuf[r, sl]
            return rc

        lax.fori_loop(0, CH, row_body, 0)
        store[t] = pltpu.async_copy(
            xbuf, out_hbm.at[b, pl.ds(soff(c), CH)], ssems[t % NXB])

    for h in store.values():
        h.wait()


def _sc_call(x, emb_weight):
    mesh = plsc.VectorSubcoreMesh(core_axis_name="c", subcore_axis_name="s")
    f = pl.kernel(
        _sc_body,
        out_type=jax.ShapeDtypeStruct((B_SC, S, D), jnp.float32),
        mesh=mesh,
        scratch_types=[
            pltpu.VMEM((CH, D), jnp.float32),
            pltpu.VMEM((CH, D), jnp.float32),
            pltpu.VMEM((CH, D), jnp.float32),
            pltpu.VMEM((CH, D), jnp.float32),
            pltpu.VMEM((CH, D), jnp.float32),
            pltpu.VMEM((CH, D), jnp.float32),
            pltpu.SemaphoreType.DMA,
            pltpu.SemaphoreType.DMA,
            pltpu.SemaphoreType.DMA,
            pltpu.SemaphoreType.DMA,
            pltpu.SemaphoreType.DMA,
            pltpu.SemaphoreType.DMA,
            pltpu.SemaphoreType.DMA,
            pltpu.SemaphoreType.DMA,
            pltpu.SemaphoreType.DMA,
            pltpu.SemaphoreType.DMA,
        ],
    )
    return f(x, emb_weight)


def _tc_add_body(x_ref, e_ref, o_ref):
    o_ref[...] = x_ref[...] + e_ref[...][None, :, :]


def _tc_call(x, emb_weight):
    grid = (S // BS_TC, B - B_SC)
    return pl.pallas_call(
        _tc_add_body,
        grid=grid,
        in_specs=[
            pl.BlockSpec((1, BS_TC, D), lambda s, b: (b + B_SC, s, 0)),
            pl.BlockSpec((BS_TC, D), lambda s, b: (s, 0)),
        ],
        out_specs=pl.BlockSpec((1, BS_TC, D), lambda s, b: (b + B_SC, s, 0)),
        out_shape=jax.ShapeDtypeStruct((B, S, D), x.dtype),
    )(x, emb_weight)


def _merge_body(sc_ref, tc_ref, o_ref):
    o_ref[...] = sc_ref[...]


def _merge(tc_full, sc_out):
    return pl.pallas_call(
        _merge_body,
        grid=(S // BS_TC,),
        in_specs=[
            pl.BlockSpec((B_SC, BS_TC, D), lambda s: (0, s, 0)),
            pl.BlockSpec(memory_space=pl.ANY),
        ],
        out_specs=pl.BlockSpec((B_SC, BS_TC, D), lambda s: (0, s, 0)),
        out_shape=jax.ShapeDtypeStruct((B, S, D), tc_full.dtype),
        input_output_aliases={1: 0},
    )(sc_out, tc_full)


def kernel(x, emb_weight):
    sc_out = _sc_call(x, emb_weight)
    tc_out = _tc_call(x, emb_weight)
    return _merge(tc_out, sc_out)


# speedup vs baseline: 1.1673x; 1.0450x over previous
import jax
import jax.numpy as jnp
from jax import lax
from jax.experimental import pallas as pl
from jax.experimental.pallas import tpu as pltpu
from jax.experimental.pallas import tpu_sc as plsc

B = 4
B_SC = 1
S = 2048
D = 1024
NC = 2
NS = 16
NW = NC * NS
SPW = S // NW
CH = 16
NCHUNK = SPW // CH
NSTAGE = NCHUNK * B_SC
NVEC = D // 16
NXB = 4
BS_TC = 1024


def _sc_body(x_hbm, emb_hbm, out_hbm,
             eb0, eb1, xb0, xb1, xb2, xb3,
             esem0, esem1, ls0, ls1, ls2, ls3, ss0, ss1, ss2, ss3):
    wid = lax.axis_index("s") * NC + lax.axis_index("c")
    s0 = wid * SPW
    ebufs, esems = (eb0, eb1), (esem0, esem1)
    xbufs = (xb0, xb1, xb2, xb3)
    lsems = (ls0, ls1, ls2, ls3)
    ssems = (ss0, ss1, ss2, ss3)

    def soff(c):
        return s0 + c * CH

    def start_load(t):
        c, b = divmod(t, B_SC)
        return pltpu.async_copy(
            x_hbm.at[b, pl.ds(soff(c), CH)], xbufs[t % NXB], lsems[t % NXB])

    eload = {0: pltpu.async_copy(emb_hbm.at[pl.ds(soff(0), CH)], eb0, esem0)}
    xload = {t: start_load(t) for t in range(min(2, NSTAGE))}
    store = {}

    for t in range(NSTAGE):
        c, b = divmod(t, B_SC)
        if b == 0 and c + 1 < NCHUNK:
            ne = (c + 1) % 2
            eload[c + 1] = pltpu.async_copy(
                emb_hbm.at[pl.ds(soff(c + 1), CH)], ebufs[ne], esems[ne])
        if t + 2 < NSTAGE:
            if t - 2 in store:
                store.pop(t - 2).wait()
            xload[t + 2] = start_load(t + 2)
        xload.pop(t).wait()
        if b == 0:
            eload.pop(c).wait()

        xbuf, ebuf = xbufs[t % NXB], ebufs[c % 2]

        def row_body(r, rc, xbuf=xbuf, ebuf=ebuf):
            for j in range(NVEC):
                sl = pl.ds(j * 16, 16)
                xbuf[r, sl] = xbuf[r, sl] + ebuf[r, sl]
            return rc

        lax.fori_loop(0, CH, row_body, 0)
        store[t] = pltpu.async_copy(
            xbuf, out_hbm.at[b, pl.ds(soff(c), CH)], ssems[t % NXB])

    for h in store.values():
        h.wait()


def _sc_call(x, emb_weight):
    mesh = plsc.VectorSubcoreMesh(core_axis_name="c", subcore_axis_name="s")
    f = pl.kernel(
        _sc_body,
        out_type=jax.ShapeDtypeStruct((B_SC, S, D), jnp.float32),
        mesh=mesh,
        scratch_types=[
            pltpu.VMEM((CH, D), jnp.float32),
            pltpu.VMEM((CH, D), jnp.float32),
            pltpu.VMEM((CH, D), jnp.float32),
            pltpu.VMEM((CH, D), jnp.float32),
            pltpu.VMEM((CH, D), jnp.float32),
            pltpu.VMEM((CH, D), jnp.float32),
            pltpu.SemaphoreType.DMA,
            pltpu.SemaphoreType.DMA,
            pltpu.SemaphoreType.DMA,
            pltpu.SemaphoreType.DMA,
            pltpu.SemaphoreType.DMA,
            pltpu.SemaphoreType.DMA,
            pltpu.SemaphoreType.DMA,
            pltpu.SemaphoreType.DMA,
            pltpu.SemaphoreType.DMA,
            pltpu.SemaphoreType.DMA,
        ],
    )
    return f(x, emb_weight)


def _tc_add_body(x_ref, e_ref, o_ref):
    o_ref[...] = x_ref[...] + e_ref[...][None, :, :]


def _tc_call(x, emb_weight):
    grid = (S // BS_TC, B - B_SC)
    return pl.pallas_call(
        _tc_add_body,
        grid=grid,
        in_specs=[
            pl.BlockSpec((1, BS_TC, D), lambda s, b: (b + B_SC, s, 0)),
            pl.BlockSpec((BS_TC, D), lambda s, b: (s, 0)),
        ],
        out_specs=pl.BlockSpec((1, BS_TC, D), lambda s, b: (b + B_SC, s, 0)),
        out_shape=jax.ShapeDtypeStruct((B, S, D), x.dtype),
    )(x, emb_weight)


def _merge_body(sc_ref, tc_ref, o_ref):
    o_ref[...] = sc_ref[...]


def _merge(tc_full, sc_out):
    return pl.pallas_call(
        _merge_body,
        grid=(S // BS_TC,),
        in_specs=[
            pl.BlockSpec((B_SC, BS_TC, D), lambda s: (0, s, 0)),
            pl.BlockSpec(memory_space=pl.ANY),
        ],
        out_specs=pl.BlockSpec((B_SC, BS_TC, D), lambda s: (0, s, 0)),
        out_shape=jax.ShapeDtypeStruct((B, S, D), tc_full.dtype),
        input_output_aliases={1: 0},
    )(sc_out, tc_full)


def kernel(x, emb_weight):
    sc_out = _sc_call(x, emb_weight)
    tc_out = _tc_call(x, emb_weight)
    return _merge(tc_out, sc_out)


# speedup vs baseline: 1.1694x; 1.0018x over previous
import jax
import jax.numpy as jnp
from jax import lax
from jax.experimental import pallas as pl
from jax.experimental.pallas import tpu as pltpu
from jax.experimental.pallas import tpu_sc as plsc

B = 4
B_SC = 1
S = 2048
D = 1024
NC = 2
NS = 16
NW = NC * NS
SPW = S // NW
CH = 16
NCHUNK = SPW // CH
NSTAGE = NCHUNK * B_SC
NVEC = D // 16
NXB = 4
BS_TC = 2048


def _sc_body(x_hbm, emb_hbm, out_hbm,
             eb0, eb1, xb0, xb1, xb2, xb3,
             esem0, esem1, ls0, ls1, ls2, ls3, ss0, ss1, ss2, ss3):
    wid = lax.axis_index("s") * NC + lax.axis_index("c")
    s0 = wid * SPW
    ebufs, esems = (eb0, eb1), (esem0, esem1)
    xbufs = (xb0, xb1, xb2, xb3)
    lsems = (ls0, ls1, ls2, ls3)
    ssems = (ss0, ss1, ss2, ss3)

    def soff(c):
        return s0 + c * CH

    def start_load(t):
        c, b = divmod(t, B_SC)
        return pltpu.async_copy(
            x_hbm.at[b, pl.ds(soff(c), CH)], xbufs[t % NXB], lsems[t % NXB])

    eload = {0: pltpu.async_copy(emb_hbm.at[pl.ds(soff(0), CH)], eb0, esem0)}
    xload = {t: start_load(t) for t in range(min(2, NSTAGE))}
    store = {}

    for t in range(NSTAGE):
        c, b = divmod(t, B_SC)
        if b == 0 and c + 1 < NCHUNK:
            ne = (c + 1) % 2
            eload[c + 1] = pltpu.async_copy(
                emb_hbm.at[pl.ds(soff(c + 1), CH)], ebufs[ne], esems[ne])
        if t + 2 < NSTAGE:
            if t - 2 in store:
                store.pop(t - 2).wait()
            xload[t + 2] = start_load(t + 2)
        xload.pop(t).wait()
        if b == 0:
            eload.pop(c).wait()

        xbuf, ebuf = xbufs[t % NXB], ebufs[c % 2]

        def row_body(r, rc, xbuf=xbuf, ebuf=ebuf):
            for j in range(NVEC):
                sl = pl.ds(j * 16, 16)
                xbuf[r, sl] = xbuf[r, sl] + ebuf[r, sl]
            return rc

        lax.fori_loop(0, CH, row_body, 0)
        store[t] = pltpu.async_copy(
            xbuf, out_hbm.at[b, pl.ds(soff(c), CH)], ssems[t % NXB])

    for h in store.values():
        h.wait()


def _sc_call(x, emb_weight):
    mesh = plsc.VectorSubcoreMesh(core_axis_name="c", subcore_axis_name="s")
    f = pl.kernel(
        _sc_body,
        out_type=jax.ShapeDtypeStruct((B_SC, S, D), jnp.float32),
        mesh=mesh,
        scratch_types=[
            pltpu.VMEM((CH, D), jnp.float32),
            pltpu.VMEM((CH, D), jnp.float32),
            pltpu.VMEM((CH, D), jnp.float32),
            pltpu.VMEM((CH, D), jnp.float32),
            pltpu.VMEM((CH, D), jnp.float32),
            pltpu.VMEM((CH, D), jnp.float32),
            pltpu.SemaphoreType.DMA,
            pltpu.SemaphoreType.DMA,
            pltpu.SemaphoreType.DMA,
            pltpu.SemaphoreType.DMA,
            pltpu.SemaphoreType.DMA,
            pltpu.SemaphoreType.DMA,
            pltpu.SemaphoreType.DMA,
            pltpu.SemaphoreType.DMA,
            pltpu.SemaphoreType.DMA,
            pltpu.SemaphoreType.DMA,
        ],
    )
    return f(x, emb_weight)


def _tc_add_body(x_ref, e_ref, o_ref):
    o_ref[...] = x_ref[...] + e_ref[...][None, :, :]


def _tc_call(x, emb_weight):
    grid = (S // BS_TC, B - B_SC)
    return pl.pallas_call(
        _tc_add_body,
        grid=grid,
        in_specs=[
            pl.BlockSpec((1, BS_TC, D), lambda s, b: (b + B_SC, s, 0)),
            pl.BlockSpec((BS_TC, D), lambda s, b: (s, 0)),
        ],
        out_specs=pl.BlockSpec((1, BS_TC, D), lambda s, b: (b + B_SC, s, 0)),
        out_shape=jax.ShapeDtypeStruct((B, S, D), x.dtype),
    )(x, emb_weight)


def _merge_body(sc_ref, tc_ref, o_ref):
    o_ref[...] = sc_ref[...]


def _merge(tc_full, sc_out):
    return pl.pallas_call(
        _merge_body,
        grid=(S // BS_TC,),
        in_specs=[
            pl.BlockSpec((B_SC, BS_TC, D), lambda s: (0, s, 0)),
            pl.BlockSpec(memory_space=pl.ANY),
        ],
        out_specs=pl.BlockSpec((B_SC, BS_TC, D), lambda s: (0, s, 0)),
        out_shape=jax.ShapeDtypeStruct((B, S, D), tc_full.dtype),
        input_output_aliases={1: 0},
    )(sc_out, tc_full)


def kernel(x, emb_weight):
    sc_out = _sc_call(x, emb_weight)
    tc_out = _tc_call(x, emb_weight)
    return _merge(tc_out, sc_out)
